# baseline (speedup 1.0000x reference)
import functools

import jax
import jax.numpy as jnp
from jax import lax
from jax.experimental import pallas as pl
from jax.experimental.pallas import tpu as pltpu
from jax.experimental.pallas import tpu_sc as plsc

N_TRACKS = 1000000
DIM_TRACK = 64
BATCH = 4096
HIST = 50

NC = 2
NS = 16
NW = NC * NS
TOTAL = BATCH * HIST
PER_W = TOTAL // NW
CHUNK = 128
NSTEP = PER_W // CHUNK
GROUP = 5
NGROUP = NSTEP // GROUP
ROWS = GROUP * CHUNK


def _emb_kernel(idx_hbm, table_hbm, out_hbm, idx_v, rows_v, gs0, gs1, os0, os1):
    wid = lax.axis_index("s") * NC + lax.axis_index("c")
    out_base = wid * PER_W

    pltpu.sync_copy(idx_hbm.at[wid], idx_v)

    gsems = (gs0, gs1)
    osems = (os0, os1)

    @pl.loop(0, NGROUP, step=2)
    def _group_pair(go):
        for p in range(2):
            g = go + p
            buf = rows_v.at[p]

            @pl.when(g >= 2)
            def _wait_prev():
                pltpu.make_async_copy(
                    buf, out_hbm.at[pl.ds(0, ROWS)], osems[p]
                ).wait()

            for c in range(GROUP):
                pltpu.async_copy(
                    table_hbm.at[idx_v.at[g * GROUP + c]],
                    buf.at[pl.ds(c * CHUNK, CHUNK)],
                    gsems[p],
                )
            for c in range(GROUP):
                pltpu.make_async_copy(
                    table_hbm.at[idx_v.at[g * GROUP + c]],
                    buf.at[pl.ds(c * CHUNK, CHUNK)],
                    gsems[p],
                ).wait()

            pltpu.async_copy(
                buf, out_hbm.at[pl.ds(out_base + g * ROWS, ROWS)], osems[p]
            )

    for p in range(2):
        pltpu.make_async_copy(
            rows_v.at[p], out_hbm.at[pl.ds(0, ROWS)], osems[p]
        ).wait()


@jax.jit
def kernel(indices_, table):
    idx = indices_.reshape(NW, NSTEP, CHUNK)
    mesh = plsc.VectorSubcoreMesh(
        core_axis_name="c", subcore_axis_name="s", num_cores=NC, num_subcores=NS
    )
    out = pl.kernel(
        _emb_kernel,
        out_type=jax.ShapeDtypeStruct((TOTAL, DIM_TRACK), jnp.float32),
        mesh=mesh,
        scratch_types=[
            pltpu.VMEM((NSTEP, CHUNK), jnp.int32),
            pltpu.VMEM((2, ROWS, DIM_TRACK), jnp.float32),
            pltpu.SemaphoreType.DMA,
            pltpu.SemaphoreType.DMA,
            pltpu.SemaphoreType.DMA,
            pltpu.SemaphoreType.DMA,
        ],
        compiler_params=pltpu.CompilerParams(use_tc_tiling_on_sc=False),
    )(idx, table)
    return out.reshape(BATCH, HIST, DIM_TRACK)
